# baseline (speedup 1.0000x reference)
import functools

import jax
import jax.numpy as jnp
from jax import lax
from jax.experimental import pallas as pl
from jax.experimental.pallas import tpu as pltpu
from jax.experimental.pallas import tpu_sc as plsc

N = 10000
E = 320000
D = 128
C = 2

NC = 2
NS = 16
NW = NC * NS
EC = 80
EPT = E // NW
NCHUNK = EPT // EC
DGC = 64
DGT = 160
DGPAD = NW * DGT * DGC - E
NP = 10240
RPT = NP // NS


_mesh = plsc.VectorSubcoreMesh(core_axis_name="c", subcore_axis_name="s")


@functools.partial(
    pl.kernel,
    mesh=_mesh,
    out_type=jax.ShapeDtypeStruct((NC, NP), jnp.float32),
    scratch_types=[
        pltpu.VMEM((DGT, DGC), jnp.int32),
        pltpu.VMEM((DGC,), jnp.float32),
        pltpu.VMEM((RPT,), jnp.float32),
        pltpu.VMEM_SHARED((NP,), jnp.float32),
    ],
)
def _deg(dstp_hbm, out_hbm, dst_v, ones_v, zbuf_v, acc_sh):
    cid = lax.axis_index("c")
    sid = lax.axis_index("s")
    wid = cid * NS + sid

    one16 = jnp.ones((16,), jnp.float32)
    for j in range(DGC // 16):
        ones_v[pl.ds(j * 16, 16)] = one16
    z16 = jnp.zeros((16,), jnp.float32)

    def zb(i, carry):
        zbuf_v[pl.ds(i * 16, 16)] = z16
        return carry

    lax.fori_loop(0, RPT // 16, zb, 0)
    pltpu.sync_copy(zbuf_v, acc_sh.at[pl.ds(sid * RPT, RPT)])
    pltpu.sync_copy(dstp_hbm.at[wid], dst_v)
    plsc.subcore_barrier()

    def chunk(j, carry):
        pltpu.sync_copy(ones_v, acc_sh.at[dst_v.at[j]], add=True)
        return carry

    lax.fori_loop(0, DGT, chunk, 0)
    plsc.subcore_barrier()
    pltpu.sync_copy(acc_sh.at[pl.ds(sid * RPT, RPT)],
                    out_hbm.at[cid, pl.ds(sid * RPT, RPT)])


@functools.partial(
    pl.kernel,
    mesh=_mesh,
    out_type=jax.ShapeDtypeStruct((NC, NP, D), jnp.float32),
    scratch_types=[
        pltpu.VMEM((EC,), jnp.int32),
        pltpu.VMEM((EC,), jnp.int32),
        pltpu.VMEM((EC,), jnp.int32),
        pltpu.VMEM((EC,), jnp.int32),
        pltpu.VMEM((EC,), jnp.int32),
        pltpu.VMEM((EC,), jnp.int32),
        pltpu.VMEM((EC, D), jnp.float32),
        pltpu.VMEM((EC, D), jnp.float32),
        pltpu.VMEM((EC, D), jnp.float32),
        pltpu.VMEM_SHARED((NP, D), jnp.float32),
        pltpu.SemaphoreType.DMA,
        pltpu.SemaphoreType.DMA,
        pltpu.SemaphoreType.DMA,
        pltpu.SemaphoreType.DMA,
        pltpu.SemaphoreType.DMA,
        pltpu.SemaphoreType.DMA,
        pltpu.SemaphoreType.DMA,
        pltpu.SemaphoreType.DMA,
        pltpu.SemaphoreType.DMA,
    ],
)
def _agg(hs_hbm, src_hbm, dst_hbm, out_hbm, src_0, src_1, src_2, dst_0,
         dst_1, dst_2, rows_0, rows_1, rows_2, acc_sh, gs_0, gs_1, gs_2,
         ss_0, ss_1, ss_2, ds_0, ds_1, ds_2):
    srcs = (src_0, src_1, src_2)
    dsts = (dst_0, dst_1, dst_2)
    rows = (rows_0, rows_1, rows_2)
    gsem = (gs_0, gs_1, gs_2)
    ssem = (ss_0, ss_1, ss_2)
    dsem = (ds_0, ds_1, ds_2)
    cid = lax.axis_index("c")
    sid = lax.axis_index("s")
    wid = cid * NS + sid

    z16 = jnp.zeros((16,), jnp.float32)

    def zrow(i, carry):
        for j in range(D // 16):
            rows_0[i, pl.ds(j * 16, 16)] = z16
        return carry

    lax.fori_loop(0, EC, zrow, 0)

    row0 = sid * RPT
    for r in range(RPT // EC):
        pltpu.sync_copy(rows_0, acc_sh.at[pl.ds(row0 + r * EC, EC)])
    plsc.subcore_barrier()

    ebase = wid * EPT

    def idx_start(c, b):
        off = ebase + c * EC
        pltpu.async_copy(src_hbm.at[pl.ds(off, EC)], srcs[b], ssem[b])
        pltpu.async_copy(dst_hbm.at[pl.ds(off, EC)], dsts[b], dsem[b])

    def idx_wait(c, b):
        off = ebase + c * EC
        pltpu.make_async_copy(src_hbm.at[pl.ds(off, EC)], srcs[b],
                              ssem[b]).wait()
        pltpu.make_async_copy(dst_hbm.at[pl.ds(off, EC)], dsts[b],
                              dsem[b]).wait()

    def gather_start(b):
        pltpu.async_copy(hs_hbm.at[srcs[b]], rows[b], gsem[b])

    def gather_wait(b):
        pltpu.make_async_copy(hs_hbm.at[srcs[b]], rows[b], gsem[b]).wait()

    def scatter(b):
        pltpu.sync_copy(rows[b], acc_sh.at[dsts[b]], add=True)

    idx_start(0, 0)
    idx_start(1, 1)
    idx_wait(0, 0)
    gather_start(0)

    def group(g, carry):
        for b in range(3):
            c = 3 * g + b
            gather_wait(b)
            scatter(b)
            idx_start(c + 2, (b + 2) % 3)
            idx_wait(c + 1, (b + 1) % 3)
            gather_start((b + 1) % 3)
        return carry

    lax.fori_loop(0, NCHUNK // 3, group, 0)
    gather_wait(0)
    scatter(0)
    idx_wait(NCHUNK - 1, 1)
    gather_start(1)
    gather_wait(1)
    scatter(1)

    plsc.subcore_barrier()
    pltpu.sync_copy(acc_sh.at[pl.ds(row0, RPT)],
                    out_hbm.at[cid, pl.ds(row0, RPT)])


BLK = 1000
GRID = N // BLK
_HI = lax.Precision.HIGHEST


def _tc1_body(degT_ref, x_ref, w1_ref, dis_ref, hs1_ref):
    deg = degT_ref[...]
    dis = lax.rsqrt(deg[:, 0:1] + deg[:, 1:2] + 1.0)
    dis_ref[...] = dis
    mm = jnp.dot(x_ref[...], w1_ref[...], preferred_element_type=jnp.float32,
                 precision=_HI)
    hs1_ref[...] = mm * dis


_tc1 = pl.pallas_call(
    _tc1_body,
    grid=(GRID,),
    in_specs=[
        pl.BlockSpec((BLK, NC), lambda i: (i, 0)),
        pl.BlockSpec((BLK, D), lambda i: (i, 0)),
        pl.BlockSpec((D, D), lambda i: (0, 0)),
    ],
    out_specs=[
        pl.BlockSpec((BLK, 1), lambda i: (i, 0)),
        pl.BlockSpec((BLK, D), lambda i: (i, 0)),
    ],
    out_shape=[
        jax.ShapeDtypeStruct((N, 1), jnp.float32),
        jax.ShapeDtypeStruct((N, D), jnp.float32),
    ],
)


def _tc2_body(a0_ref, a1_ref, hs1_ref, dis_ref, b1_ref, w2_ref, hs2_ref):
    dis = dis_ref[...]
    t = a0_ref[...] + a1_ref[...] + hs1_ref[...]
    h1 = jnp.maximum(dis * t + b1_ref[...], 0.0)
    hs2_ref[...] = jnp.dot(h1, w2_ref[...], preferred_element_type=jnp.float32,
                           precision=_HI) * dis


_tc2 = pl.pallas_call(
    _tc2_body,
    grid=(GRID,),
    in_specs=[
        pl.BlockSpec((BLK, D), lambda i: (i, 0)),
        pl.BlockSpec((BLK, D), lambda i: (i, 0)),
        pl.BlockSpec((BLK, D), lambda i: (i, 0)),
        pl.BlockSpec((BLK, 1), lambda i: (i, 0)),
        pl.BlockSpec((1, D), lambda i: (0, 0)),
        pl.BlockSpec((D, D), lambda i: (0, 0)),
    ],
    out_specs=pl.BlockSpec((BLK, D), lambda i: (i, 0)),
    out_shape=jax.ShapeDtypeStruct((N, D), jnp.float32),
)


def _tc3_body(a0_ref, a1_ref, hs2_ref, dis_ref, b2_ref, wc_ref, bc_ref,
              logits_ref, h2_ref):
    dis = dis_ref[...]
    t = a0_ref[...] + a1_ref[...] + hs2_ref[...]
    h2 = jnp.maximum(dis * t + b2_ref[...], 0.0)
    h2_ref[...] = h2
    logits_ref[...] = jnp.dot(h2, wc_ref[...],
                              preferred_element_type=jnp.float32,
                              precision=_HI) + bc_ref[...]


_tc3 = pl.pallas_call(
    _tc3_body,
    grid=(GRID,),
    in_specs=[
        pl.BlockSpec((BLK, D), lambda i: (i, 0)),
        pl.BlockSpec((BLK, D), lambda i: (i, 0)),
        pl.BlockSpec((BLK, D), lambda i: (i, 0)),
        pl.BlockSpec((BLK, 1), lambda i: (i, 0)),
        pl.BlockSpec((1, D), lambda i: (0, 0)),
        pl.BlockSpec((D, C), lambda i: (0, 0)),
        pl.BlockSpec((1, C), lambda i: (0, 0)),
    ],
    out_specs=[
        pl.BlockSpec((BLK, C), lambda i: (i, 0)),
        pl.BlockSpec((BLK, D), lambda i: (i, 0)),
    ],
    out_shape=[
        jax.ShapeDtypeStruct((N, C), jnp.float32),
        jax.ShapeDtypeStruct((N, D), jnp.float32),
    ],
)


def kernel(x, edge_index, W1, b1, W2, b2, Wc, bc):
    ei = edge_index.astype(jnp.int32)
    src = ei[0]
    dst = ei[1]
    pad_dst = N + (jnp.arange(DGPAD, dtype=jnp.int32) % (NP - N))
    dstp = jnp.concatenate([dst, pad_dst]).reshape(NW, DGT, DGC)
    degp = _deg(dstp)
    dis, hs1 = _tc1(degp[:, :N].T, x, W1)
    agg1 = _agg(hs1, src, dst)
    hs2 = _tc2(agg1[0, :N], agg1[1, :N], hs1, dis, b1.reshape(1, D), W2)
    agg2 = _agg(hs2, src, dst)
    logits, h2 = _tc3(agg2[0, :N], agg2[1, :N], hs2, dis, b2.reshape(1, D),
                      Wc, bc.reshape(1, C))
    return (logits, h2)
